# baseline (speedup 1.0000x reference)
import functools

import jax
import jax.numpy as jnp
from jax import lax
from jax.experimental import pallas as pl
from jax.experimental.pallas import tpu as pltpu
from jax.experimental.pallas import tpu_sc as plsc

BATCH = 16384
FIELDS = 26
NUM_NODES = 32
NUM_CATS = 1000000

NW = 32
NTJ = BATCH // 128
NGROUP = FIELDS * NTJ
G_PER_W = NGROUP // NW
NBUF = 2
NTI = NUM_NODES // 8

_mesh = plsc.VectorSubcoreMesh(core_axis_name="c", subcore_axis_name="s")


@functools.partial(
    pl.kernel,
    mesh=_mesh,
    compiler_params=pltpu.CompilerParams(use_tc_tiling_on_sc=False, needs_layout_passes=False),
    out_type=jax.ShapeDtypeStruct((FIELDS, NTI, NTJ, 8, 128), jnp.float32),
    scratch_types=[
        pltpu.VMEM((NBUF, 128), jnp.int32),
        pltpu.VMEM((NBUF, 128, NUM_NODES), jnp.float32),
        pltpu.VMEM((NBUF, NTI, 8, 128), jnp.float32),
        pltpu.SemaphoreType.DMA,
        pltpu.SemaphoreType.DMA,
        pltpu.SemaphoreType.DMA,
        pltpu.SemaphoreType.DMA,
        pltpu.SemaphoreType.DMA,
    ],
)
def _gather_rows(idxt_hbm, table_hbm, out_hbm, idx_v, rows_v, tile_v,
                 sem_l0, sem_l1, sem_s0, sem_s1, sem_g):
    wid = lax.axis_index("s") * 2 + lax.axis_index("c")
    gbase = wid * G_PER_W
    sem_l = (sem_l0, sem_l1)
    sem_s = (sem_s0, sem_s1)
    lanes = lax.iota(jnp.int32, 16)
    ti_lo = lanes // 8
    ti_hi = ti_lo + 2
    s_all = lax.rem(lanes, 8)

    def fg(g):
        return g // NTJ, lax.rem(g, NTJ)

    def idx_src(g):
        f, tj = fg(g)
        return idxt_hbm.at[f].at[pl.ds(pl.multiple_of(tj * 128, 128), 128)]

    def store_copies(b, g, sem):
        f, tj = fg(g)
        return [
            pltpu.make_async_copy(
                tile_v.at[b].at[ti], out_hbm.at[f].at[ti].at[tj], sem)
            for ti in range(NTI)
        ]

    for b in range(NBUF):
        pltpu.async_copy(idx_src(gbase + b), idx_v.at[b], sem_l[b])

    def body(i, carry):
        g0 = gbase + i * NBUF
        for b in range(NBUF):
            g = g0 + b
            pltpu.make_async_copy(idx_src(g), idx_v.at[b], sem_l[b]).wait()
            pltpu.async_copy(table_hbm.at[idx_v.at[b]], rows_v.at[b],
                             sem_g).wait()
            @pl.when(i * NBUF + b + NBUF < G_PER_W)
            def _():
                pltpu.async_copy(idx_src(g + NBUF), idx_v.at[b], sem_l[b])
            @pl.when(i >= 1)
            def _():
                for cp in store_copies(b, g - NBUF, sem_s[b]):
                    cp.wait()
            def trow(r, carry):
                cvec = jnp.full((16,), 1, jnp.int32) * r
                va = rows_v[b, r, pl.ds(0, 16)]
                vb = rows_v[b, r, pl.ds(16, 16)]
                plsc.store_scatter(tile_v.at[b], [ti_lo, s_all, cvec], va)
                plsc.store_scatter(tile_v.at[b], [ti_hi, s_all, cvec], vb)
                return carry
            lax.fori_loop(0, 128, trow, 0, unroll=8)
            for cp in store_copies(b, g, sem_s[b]):
                cp.start()
        return carry

    lax.fori_loop(0, G_PER_W // NBUF, body, 0, unroll=False)

    for b in range(NBUF):
        g = gbase + G_PER_W - NBUF + b
        for cp in store_copies(b, g, sem_s[b]):
            cp.wait()


def kernel(indices, E):
    idxt = indices.T.astype(jnp.int32)
    y5 = _gather_rows(idxt, E)
    return y5.transpose(2, 4, 0, 1, 3).reshape(BATCH, FIELDS, NUM_NODES)


# speedup vs baseline: 1.1068x; 1.1068x over previous
import functools

import jax
import jax.numpy as jnp
from jax import lax
from jax.experimental import pallas as pl
from jax.experimental.pallas import tpu as pltpu
from jax.experimental.pallas import tpu_sc as plsc

BATCH = 16384
FIELDS = 26
NUM_NODES = 32
NUM_CATS = 1000000

NW = 32
NTJ = BATCH // 128
NGROUP = FIELDS * NTJ
G_PER_W = NGROUP // NW
NBUF = 2
NTI = NUM_NODES // 8

_mesh = plsc.VectorSubcoreMesh(core_axis_name="c", subcore_axis_name="s")


@functools.partial(
    pl.kernel,
    mesh=_mesh,
    compiler_params=pltpu.CompilerParams(use_tc_tiling_on_sc=False,
                                         needs_layout_passes=False),
    out_type=jax.ShapeDtypeStruct((FIELDS, NTI, NTJ, 8, 128), jnp.float32),
    scratch_types=[
        pltpu.VMEM((NBUF, 128), jnp.int32),
        pltpu.VMEM((NBUF, 128, NUM_NODES), jnp.float32),
        pltpu.VMEM((NBUF, NTI, 8, 128), jnp.float32),
        pltpu.SemaphoreType.DMA,
        pltpu.SemaphoreType.DMA,
        pltpu.SemaphoreType.DMA,
        pltpu.SemaphoreType.DMA,
        pltpu.SemaphoreType.DMA,
    ],
)
def _gather_rows(idxt_hbm, table_hbm, out_hbm, idx_v, rows_v, tile_v,
                 sem_l0, sem_l1, sem_s0, sem_s1, sem_g):
    wid = lax.axis_index("s") * 2 + lax.axis_index("c")
    gbase = wid * G_PER_W
    gend = gbase + G_PER_W
    sem_l = (sem_l0, sem_l1)
    sem_s = (sem_s0, sem_s1)
    lanes = lax.iota(jnp.int32, 16)
    ti_lo = lanes // 8
    ti_hi = ti_lo + 2
    s_all = lax.rem(lanes, 8)

    def fg(g):
        return g // NTJ, lax.rem(g, NTJ)

    def idx_src(g):
        f, tj = fg(g)
        return idxt_hbm.at[f].at[pl.ds(pl.multiple_of(tj * 128, 128), 128)]

    def store_copies(b, g, sem):
        f, tj = fg(g)
        return [
            pltpu.make_async_copy(
                tile_v.at[b].at[ti], out_hbm.at[f].at[ti].at[tj], sem)
            for ti in range(NTI)
        ]

    def transpose_group(b):
        def trow(r, carry):
            cvec = jnp.full((16,), 1, jnp.int32) * r
            va = rows_v[b, r, pl.ds(0, 16)]
            vb = rows_v[b, r, pl.ds(16, 16)]
            plsc.store_scatter(tile_v.at[b], [ti_lo, s_all, cvec], va)
            plsc.store_scatter(tile_v.at[b], [ti_hi, s_all, cvec], vb)
            return carry
        lax.fori_loop(0, 128, trow, 0, unroll=8)

    for b in range(NBUF):
        pltpu.async_copy(idx_src(gbase + b), idx_v.at[b], sem_l[b])
    pltpu.make_async_copy(idx_src(gbase), idx_v.at[0], sem_l[0]).wait()
    pltpu.async_copy(table_hbm.at[idx_v.at[0]], rows_v.at[0], sem_g)

    def body(i, carry):
        g0 = gbase + i * NBUF
        for b in range(NBUF):
            g = g0 + b
            o = 1 - b
            pltpu.make_async_copy(table_hbm.at[idx_v.at[b]], rows_v.at[b],
                                  sem_g).wait()
            @pl.when(g + 1 < gend)
            def _():
                pltpu.make_async_copy(idx_src(g + 1), idx_v.at[o],
                                      sem_l[o]).wait()
                pltpu.async_copy(table_hbm.at[idx_v.at[o]], rows_v.at[o],
                                sem_g)
            @pl.when(g + 2 < gend)
            def _():
                pltpu.async_copy(idx_src(g + 2), idx_v.at[b], sem_l[b])
            @pl.when(i >= 1)
            def _():
                for cp in store_copies(b, g - NBUF, sem_s[b]):
                    cp.wait()
            transpose_group(b)
            for cp in store_copies(b, g, sem_s[b]):
                cp.start()
        return carry

    lax.fori_loop(0, G_PER_W // NBUF, body, 0, unroll=False)

    for b in range(NBUF):
        g = gend - NBUF + b
        for cp in store_copies(b, g, sem_s[b]):
            cp.wait()


def kernel(indices, E):
    idxt = indices.T.astype(jnp.int32)
    y5 = _gather_rows(idxt, E)
    return y5.transpose(2, 4, 0, 1, 3).reshape(BATCH, FIELDS, NUM_NODES)


# speedup vs baseline: 1.4043x; 1.2688x over previous
import functools

import jax
import jax.numpy as jnp
from jax import lax
from jax.experimental import pallas as pl
from jax.experimental.pallas import tpu as pltpu
from jax.experimental.pallas import tpu_sc as plsc

BATCH = 16384
FIELDS = 26
NUM_NODES = 32
NUM_CATS = 1000000

NW = 32
NTJ = BATCH // 128
NGROUP = FIELDS * NTJ
G_PER_W = NGROUP // NW
NBUF = 2
NTI = NUM_NODES // 8

_mesh = plsc.VectorSubcoreMesh(core_axis_name="c", subcore_axis_name="s")


@functools.partial(
    pl.kernel,
    mesh=_mesh,
    compiler_params=pltpu.CompilerParams(use_tc_tiling_on_sc=False,
                                         needs_layout_passes=False),
    out_type=jax.ShapeDtypeStruct((FIELDS, NTI, NTJ, 8, 128), jnp.float32),
    scratch_types=[
        pltpu.VMEM((NBUF, 128), jnp.int32),
        pltpu.VMEM((NBUF, 128, NUM_NODES), jnp.float32),
        pltpu.VMEM((NBUF, NTI, 8, 136), jnp.float32),
        pltpu.SemaphoreType.DMA,
        pltpu.SemaphoreType.DMA,
        pltpu.SemaphoreType.DMA,
        pltpu.SemaphoreType.DMA,
        pltpu.SemaphoreType.DMA,
    ],
)
def _gather_rows(idxt_hbm, table_hbm, out_hbm, idx_v, rows_v, tile_v,
                 sem_l0, sem_l1, sem_s0, sem_s1, sem_g):
    wid = lax.axis_index("s") * 2 + lax.axis_index("c")
    gbase = wid * G_PER_W
    gend = gbase + G_PER_W
    sem_l = (sem_l0, sem_l1)
    sem_s = (sem_s0, sem_s1)
    lanes = lax.iota(jnp.int32, 16)
    ti_lo = lanes // 8
    ti_hi = ti_lo + 2
    s_all = lax.rem(lanes, 8)

    def fg(g):
        return g // NTJ, lax.rem(g, NTJ)

    def idx_src(g):
        f, tj = fg(g)
        return idxt_hbm.at[f].at[pl.ds(pl.multiple_of(tj * 128, 128), 128)]

    def store_copies(b, g, sem):
        f, tj = fg(g)
        return [
            pltpu.make_async_copy(
                tile_v.at[b].at[ti].at[:, pl.ds(0, 128)],
                out_hbm.at[f].at[ti].at[tj], sem)
            for ti in range(NTI)
        ]

    def transpose_group(b):
        def trow(r, carry):
            cvec = jnp.full((16,), 1, jnp.int32) * r
            va = rows_v[b, r, pl.ds(0, 16)]
            vb = rows_v[b, r, pl.ds(16, 16)]
            plsc.store_scatter(tile_v.at[b], [ti_lo, s_all, cvec], va)
            plsc.store_scatter(tile_v.at[b], [ti_hi, s_all, cvec], vb)
            return carry
        lax.fori_loop(0, 128, trow, 0, unroll=8)

    for b in range(NBUF):
        pltpu.async_copy(idx_src(gbase + b), idx_v.at[b], sem_l[b])
    pltpu.make_async_copy(idx_src(gbase), idx_v.at[0], sem_l[0]).wait()
    pltpu.async_copy(table_hbm.at[idx_v.at[0]], rows_v.at[0], sem_g)

    def body(i, carry):
        g0 = gbase + i * NBUF
        for b in range(NBUF):
            g = g0 + b
            o = 1 - b
            pltpu.make_async_copy(table_hbm.at[idx_v.at[b]], rows_v.at[b],
                                  sem_g).wait()
            @pl.when(g + 1 < gend)
            def _():
                pltpu.make_async_copy(idx_src(g + 1), idx_v.at[o],
                                      sem_l[o]).wait()
                pltpu.async_copy(table_hbm.at[idx_v.at[o]], rows_v.at[o],
                                sem_g)
            @pl.when(g + 2 < gend)
            def _():
                pltpu.async_copy(idx_src(g + 2), idx_v.at[b], sem_l[b])
            @pl.when(i >= 1)
            def _():
                for cp in store_copies(b, g - NBUF, sem_s[b]):
                    cp.wait()
            transpose_group(b)
            for cp in store_copies(b, g, sem_s[b]):
                cp.start()
        return carry

    lax.fori_loop(0, G_PER_W // NBUF, body, 0, unroll=False)

    for b in range(NBUF):
        g = gend - NBUF + b
        for cp in store_copies(b, g, sem_s[b]):
            cp.wait()


def kernel(indices, E):
    idxt = indices.T.astype(jnp.int32)
    y5 = _gather_rows(idxt, E)
    return y5.transpose(2, 4, 0, 1, 3).reshape(BATCH, FIELDS, NUM_NODES)


# speedup vs baseline: 1.4066x; 1.0017x over previous
import functools

import jax
import jax.numpy as jnp
from jax import lax
from jax.experimental import pallas as pl
from jax.experimental.pallas import tpu as pltpu
from jax.experimental.pallas import tpu_sc as plsc

BATCH = 16384
FIELDS = 26
NUM_NODES = 32
NUM_CATS = 1000000

NW = 32
NTJ = BATCH // 128
NGROUP = FIELDS * NTJ
G_PER_W = NGROUP // NW
NBUF = 2
NTI = NUM_NODES // 8

_mesh = plsc.VectorSubcoreMesh(core_axis_name="c", subcore_axis_name="s")


@functools.partial(
    pl.kernel,
    mesh=_mesh,
    compiler_params=pltpu.CompilerParams(use_tc_tiling_on_sc=False,
                                         needs_layout_passes=False,
                                         skip_device_barrier=True),
    out_type=jax.ShapeDtypeStruct((FIELDS, NTI, NTJ, 8, 128), jnp.float32),
    scratch_types=[
        pltpu.VMEM((NBUF, 128), jnp.int32),
        pltpu.VMEM((NBUF, 128, NUM_NODES), jnp.float32),
        pltpu.VMEM((NBUF, NTI, 8, 136), jnp.float32),
        pltpu.SemaphoreType.DMA,
        pltpu.SemaphoreType.DMA,
        pltpu.SemaphoreType.DMA,
        pltpu.SemaphoreType.DMA,
        pltpu.SemaphoreType.DMA,
    ],
)
def _gather_rows(idxt_hbm, table_hbm, out_hbm, idx_v, rows_v, tile_v,
                 sem_l0, sem_l1, sem_s0, sem_s1, sem_g):
    wid = lax.axis_index("s") * 2 + lax.axis_index("c")
    gbase = wid * G_PER_W
    gend = gbase + G_PER_W
    sem_l = (sem_l0, sem_l1)
    sem_s = (sem_s0, sem_s1)
    lanes = lax.iota(jnp.int32, 16)
    ti_lo = lanes // 8
    ti_hi = ti_lo + 2
    s_all = lax.rem(lanes, 8)

    def fg(g):
        return g // NTJ, lax.rem(g, NTJ)

    def idx_src(g):
        f, tj = fg(g)
        return idxt_hbm.at[f].at[pl.ds(pl.multiple_of(tj * 128, 128), 128)]

    def store_copies(b, g, sem):
        f, tj = fg(g)
        return [
            pltpu.make_async_copy(
                tile_v.at[b].at[ti].at[:, pl.ds(0, 128)],
                out_hbm.at[f].at[ti].at[tj], sem)
            for ti in range(NTI)
        ]

    def transpose_group(b):
        def trow(r, carry):
            cvec = jnp.full((16,), 1, jnp.int32) * r
            va = rows_v[b, r, pl.ds(0, 16)]
            vb = rows_v[b, r, pl.ds(16, 16)]
            plsc.store_scatter(tile_v.at[b], [ti_lo, s_all, cvec], va)
            plsc.store_scatter(tile_v.at[b], [ti_hi, s_all, cvec], vb)
            return carry
        lax.fori_loop(0, 128, trow, 0, unroll=8)

    for b in range(NBUF):
        pltpu.async_copy(idx_src(gbase + b), idx_v.at[b], sem_l[b])
    pltpu.make_async_copy(idx_src(gbase), idx_v.at[0], sem_l[0]).wait()
    pltpu.async_copy(table_hbm.at[idx_v.at[0]], rows_v.at[0], sem_g)

    def body(i, carry):
        g0 = gbase + i * NBUF
        for b in range(NBUF):
            g = g0 + b
            o = 1 - b
            pltpu.make_async_copy(table_hbm.at[idx_v.at[b]], rows_v.at[b],
                                  sem_g).wait()
            @pl.when(g + 1 < gend)
            def _():
                pltpu.make_async_copy(idx_src(g + 1), idx_v.at[o],
                                      sem_l[o]).wait()
                pltpu.async_copy(table_hbm.at[idx_v.at[o]], rows_v.at[o],
                                sem_g)
            @pl.when(g + 2 < gend)
            def _():
                pltpu.async_copy(idx_src(g + 2), idx_v.at[b], sem_l[b])
            @pl.when(i >= 1)
            def _():
                for cp in store_copies(b, g - NBUF, sem_s[b]):
                    cp.wait()
            transpose_group(b)
            for cp in store_copies(b, g, sem_s[b]):
                cp.start()
        return carry

    lax.fori_loop(0, G_PER_W // NBUF, body, 0, unroll=False)

    for b in range(NBUF):
        g = gend - NBUF + b
        for cp in store_copies(b, g, sem_s[b]):
            cp.wait()


def kernel(indices, E):
    idxt = indices.T.astype(jnp.int32)
    y5 = _gather_rows(idxt, E)
    return y5.transpose(2, 4, 0, 1, 3).reshape(BATCH, FIELDS, NUM_NODES)
